# baseline (speedup 1.0000x reference)
import jax
import jax.numpy as jnp
from jax import lax
from jax.experimental import pallas as pl
from jax.experimental.pallas import tpu as pltpu
from jax.experimental.pallas import tpu_sc as plsc

NUM_ROWS = 1000000
DIM = 64
BATCH = 4096
HIST = 200

NC = 2
NS = 16
NW = NC * NS
LANES = 16

TOTAL = BATCH * HIST
ROWS_PER_W = TOTAL // NW
CHUNK = 128
NCHUNK = ROWS_PER_W // CHUNK
NBUF = 4


def _sc_body(table_hbm, idx_hbm, out_hbm, idx_v,
             rows_bufs, cast_bufs, gsems, osems):
    wid = lax.axis_index("s") * NC + lax.axis_index("c")
    row0 = wid * ROWS_PER_W

    pltpu.sync_copy(idx_hbm.at[pl.ds(row0, ROWS_PER_W)], idx_v)

    iota = lax.iota(jnp.int32, LANES)
    even0 = iota * 2
    odd0 = even0 + 1
    even1 = even0 + 32
    odd1 = odd0 + 32

    def gather_copy(j, b):
        return pltpu.make_async_copy(
            table_hbm.at[idx_v.at[pl.ds(j * CHUNK, CHUNK)]], rows_bufs[b],
            gsems[b])

    def out_copy(j, b):
        return pltpu.make_async_copy(
            cast_bufs[b], out_hbm.at[pl.ds(row0 + j * CHUNK, CHUNK)],
            osems[b])

    for b in range(NBUF):
        gather_copy(b, b).start()

    def outer(j0, carry):
        for b in range(NBUF):
            j = j0 * NBUF + b
            gather_copy(j, b).wait()

            @pl.when(j0 > 0)
            def _wait_prev_out():
                out_copy(j - NBUF, b).wait()

            src = rows_bufs[b]
            dst = cast_bufs[b]

            def cast_rows(r2, c, src=src, dst=dst):
                for u in range(2):
                    r = r2 * 2 + u
                    ra = jnp.full((LANES,), r, jnp.int32)
                    a0 = plsc.load_gather(src, [ra, even0])
                    b_0 = plsc.load_gather(src, [ra, odd0])
                    dst[r, pl.ds(0, 32)] = plsc.pack(
                        a0, b_0, format=plsc.PackFormat.INTERLEAVED)
                    a1 = plsc.load_gather(src, [ra, even1])
                    b_1 = plsc.load_gather(src, [ra, odd1])
                    dst[r, pl.ds(32, 32)] = plsc.pack(
                        a1, b_1, format=plsc.PackFormat.INTERLEAVED)
                return c

            lax.fori_loop(0, CHUNK // 2, cast_rows, 0)

            out_copy(j, b).start()

            @pl.when(j0 < NCHUNK // NBUF - 1)
            def _next_gather():
                gather_copy(j + NBUF, b).start()
        return carry

    lax.fori_loop(0, NCHUNK // NBUF, outer, 0)

    for b in range(NBUF):
        out_copy(NCHUNK - NBUF + b, b).wait()


@jax.jit
def _embed(indices, table):
    t = lax.optimization_barrier(table.T).T
    run = pl.kernel(
        _sc_body,
        out_type=jax.ShapeDtypeStruct((TOTAL, DIM), jnp.bfloat16),
        mesh=plsc.VectorSubcoreMesh(core_axis_name="c", subcore_axis_name="s"),
        compiler_params=pltpu.CompilerParams(
            needs_layout_passes=False, use_tc_tiling_on_sc=False),
        scratch_types=[
            pltpu.VMEM((ROWS_PER_W,), jnp.int32),
            [pltpu.VMEM((CHUNK, DIM), jnp.float32) for _ in range(NBUF)],
            [pltpu.VMEM((CHUNK, DIM), jnp.bfloat16) for _ in range(NBUF)],
            [pltpu.SemaphoreType.DMA for _ in range(NBUF)],
            [pltpu.SemaphoreType.DMA for _ in range(NBUF)],
        ],
    )
    out = run(t, indices.reshape(-1))
    return out.reshape(BATCH, HIST, DIM)


def kernel(input, embedding_weight):
    return _embed(input, embedding_weight)


# speedup vs baseline: 1.1287x; 1.1287x over previous
import jax
import jax.numpy as jnp
from jax import lax
from jax.experimental import pallas as pl
from jax.experimental.pallas import tpu as pltpu
from jax.experimental.pallas import tpu_sc as plsc

NUM_ROWS = 1000000
DIM = 64
BATCH = 4096
HIST = 200

NC = 2
NS = 16
NW = NC * NS
LANES = 16

TOTAL = BATCH * HIST
ROWS_PER_W = TOTAL // NW
CHUNK = 128
NCHUNK = ROWS_PER_W // CHUNK
NBUF = 4


def _sc_body(table_hbm, idx_hbm, out_hbm, idx_v,
             rows_bufs, cast_bufs, gsems, osems):
    wid = lax.axis_index("s") * NC + lax.axis_index("c")
    row0 = wid * ROWS_PER_W

    pltpu.sync_copy(idx_hbm.at[pl.ds(row0, ROWS_PER_W)], idx_v)

    iota = lax.iota(jnp.int32, LANES)
    even0 = iota * 2
    odd0 = even0 + 1
    even1 = even0 + 32
    odd1 = odd0 + 32

    def gather_copy(j, b):
        return pltpu.make_async_copy(
            table_hbm.at[idx_v.at[pl.ds(j * CHUNK, CHUNK)]], rows_bufs[b],
            gsems[b])

    def out_copy(j, b):
        return pltpu.make_async_copy(
            cast_bufs[b], out_hbm.at[pl.ds(row0 + j * CHUNK, CHUNK)],
            osems[b])

    for b in range(NBUF):
        gather_copy(b, b).start()

    def outer(j0, carry):
        for b in range(NBUF):
            j = j0 * NBUF + b
            gather_copy(j, b).wait()

            @pl.when(j0 > 0)
            def _wait_prev_out():
                out_copy(j - NBUF, b).wait()

            src = rows_bufs[b]
            dst = cast_bufs[b]

            @plsc.parallel_loop(0, CHUNK, unroll=8)
            def cast_row(r, src=src, dst=dst):
                ra = jnp.full((LANES,), r, jnp.int32)
                a0 = plsc.load_gather(src, [ra, even0])
                b_0 = plsc.load_gather(src, [ra, odd0])
                dst[r, pl.ds(0, 32)] = plsc.pack(
                    a0, b_0, format=plsc.PackFormat.INTERLEAVED)
                a1 = plsc.load_gather(src, [ra, even1])
                b_1 = plsc.load_gather(src, [ra, odd1])
                dst[r, pl.ds(32, 32)] = plsc.pack(
                    a1, b_1, format=plsc.PackFormat.INTERLEAVED)

            out_copy(j, b).start()

            @pl.when(j0 < NCHUNK // NBUF - 1)
            def _next_gather():
                gather_copy(j + NBUF, b).start()
        return carry

    lax.fori_loop(0, NCHUNK // NBUF, outer, 0)

    for b in range(NBUF):
        out_copy(NCHUNK - NBUF + b, b).wait()


@jax.jit
def _embed(indices, table):
    run = pl.kernel(
        _sc_body,
        out_type=jax.ShapeDtypeStruct((TOTAL, DIM), jnp.bfloat16),
        mesh=plsc.VectorSubcoreMesh(core_axis_name="c", subcore_axis_name="s"),
        compiler_params=pltpu.CompilerParams(
            needs_layout_passes=False, use_tc_tiling_on_sc=False),
        scratch_types=[
            pltpu.VMEM((ROWS_PER_W,), jnp.int32),
            [pltpu.VMEM((CHUNK, DIM), jnp.float32) for _ in range(NBUF)],
            [pltpu.VMEM((CHUNK, DIM), jnp.bfloat16) for _ in range(NBUF)],
            [pltpu.SemaphoreType.DMA for _ in range(NBUF)],
            [pltpu.SemaphoreType.DMA for _ in range(NBUF)],
        ],
    )
    out = run(table, indices.reshape(-1))
    return out.reshape(BATCH, HIST, DIM)


def kernel(input, embedding_weight):
    return _embed(input, embedding_weight)
